# baseline (speedup 1.0000x reference)
import functools

import numpy as np
import jax
import jax.numpy as jnp
from jax import lax
from jax.experimental import pallas as pl
from jax.experimental.pallas import tpu as pltpu
from jax.experimental.pallas import tpu_sc as plsc

_B, _N, _D = 4096, 2048, 8192
_N1, _N2 = 64, 128
_NC, _NS = 2, 16
_NW = _NC * _NS
_RPW = _B // _NW
_L = 16
_NP = _N + _L

_R = 128
_G = _B // _R


def _dft(n):
    k = np.arange(n)
    ang = -2.0 * np.pi * np.outer(k, k) / n
    return np.cos(ang).astype(np.float32), np.sin(ang).astype(np.float32)


_F1R, _F1I = _dft(_N1)
_F2R, _F2I = _dft(_N2)
_F1S = np.vstack([_F1R, _F1I])
_F1C = np.hstack([_F1R, _F1I])
_G2F = np.block([[_F2R, _F2I], [-_F2I, _F2R]])
_G2B = np.block([[_F2R, -_F2I], [_F2I, _F2R]])
_ang = -2.0 * np.pi * np.outer(np.arange(_N1), np.arange(_N2)) / _D
_TWR = np.cos(_ang).astype(np.float32)
_TWI = np.sin(_ang).astype(np.float32)


def _sc_body(v1h, v2h, hp1h, vp1h, sp1h, hp2h, vp2h, sp2h, nfh,
             sk1h, sk2h,
             hv1, hv2, vv1, vv2, sv1, sv2, nfv,
             vb0, vb1, acc0, acc1,
             semv0, semv1, sema0, sema1):
    wid = lax.axis_index("s") * _NC + lax.axis_index("c")
    base = wid * _RPW
    pltpu.sync_copy(hp1h, hv1)
    pltpu.sync_copy(hp2h, hv2)
    pltpu.sync_copy(vp1h, vv1)
    pltpu.sync_copy(vp2h, vv2)
    pltpu.sync_copy(sp1h, sv1)
    pltpu.sync_copy(sp2h, sv2)
    pltpu.sync_copy(nfh, nfv)
    vb = (vb0, vb1)
    acc = (acc0, acc1)
    semv = (semv0, semv1)
    sema = (sema0, sema1)
    zv = jnp.zeros((_L,), jnp.int32)

    for k in (0, 1):
        @pl.loop(0, _N1, unroll=4)
        def _z0(i, _k=k):
            for j in range(_N2 // _L):
                acc[_k][i, pl.ds(j * _L, _L)] = jnp.zeros((_L,), jnp.float32)

    def zero_touched(accr, hv):
        @pl.loop(0, _NP // _L, unroll=8)
        def _z(j):
            idx = hv[pl.ds(j * _L, _L)]
            plsc.store_scatter(accr, [zv, idx], jnp.zeros((_L,), jnp.float32))

    def phase(vh, skh, hv, vv, sv, nf16):
        for k in (0, 1):
            pltpu.make_async_copy(vh.at[base + k], vb[k], semv[k]).start()

        @pl.loop(0, _RPW // 2)
        def _pair(p):
            r0 = p * 2
            for k in (0, 1):
                r = r0 + k
                row = base + r

                @pl.when(r >= 2)
                def _reclaim(_k=k, _row=row):
                    pltpu.make_async_copy(
                        acc[_k],
                        skh.at[:, pl.ds((_row - 2) * _N2, _N2)],
                        sema[_k]).wait()

                pltpu.make_async_copy(vh.at[row], vb[k], semv[k]).wait()

                @pl.loop(0, nf16)
                def _first(j, _k=k):
                    idx = hv[pl.ds(j * _L, _L)]
                    vi = vv[pl.ds(j * _L, _L)]
                    val = plsc.load_gather(vb[_k], [vi]) * sv[pl.ds(j * _L, _L)]
                    plsc.store_scatter(acc[_k], [zv, idx], val)

                @pl.loop(nf16, _NP // _L)
                def _dup(j, _k=k):
                    idx = hv[pl.ds(j * _L, _L)]
                    vi = vv[pl.ds(j * _L, _L)]
                    val = plsc.load_gather(vb[_k], [vi]) * sv[pl.ds(j * _L, _L)]
                    plsc.addupdate_scatter(acc[_k], [zv, idx], val)

                pltpu.make_async_copy(acc[k],
                                      skh.at[:, pl.ds(row * _N2, _N2)],
                                      sema[k]).start()

                @pl.when(r + 2 < _RPW)
                def _prefetch(_k=k, _row=row):
                    pltpu.make_async_copy(vh.at[_row + 2], vb[_k],
                                          semv[_k]).start()

        for k in (0, 1):
            row = base + _RPW - 2 + k
            pltpu.make_async_copy(acc[k],
                                  skh.at[:, pl.ds(row * _N2, _N2)],
                                  sema[k]).wait()
            zero_touched(acc[k], hv)

    nfvec = nfv[pl.ds(0, _L)]
    phase(v1h, sk1h, hv1, vv1, sv1, nfvec[0])
    phase(v2h, sk2h, hv2, vv2, sv2, nfvec[1])


@functools.cache
def _sc_sketch():
    return pl.kernel(
        _sc_body,
        out_type=[jax.ShapeDtypeStruct((_N1, _B * _N2), jnp.float32),
                  jax.ShapeDtypeStruct((_N1, _B * _N2), jnp.float32)],
        mesh=plsc.VectorSubcoreMesh(core_axis_name="c", subcore_axis_name="s",
                                    num_cores=_NC, num_subcores=_NS),
        compiler_params=pltpu.CompilerParams(needs_layout_passes=False),
        scratch_types=[
            pltpu.VMEM((_NP,), jnp.int32),
            pltpu.VMEM((_NP,), jnp.int32),
            pltpu.VMEM((_NP,), jnp.int32),
            pltpu.VMEM((_NP,), jnp.int32),
            pltpu.VMEM((_NP,), jnp.float32),
            pltpu.VMEM((_NP,), jnp.float32),
            pltpu.VMEM((_L,), jnp.int32),
            pltpu.VMEM((_N,), jnp.float32),
            pltpu.VMEM((_N,), jnp.float32),
            pltpu.VMEM((_N1, _N2), jnp.float32),
            pltpu.VMEM((_N1, _N2), jnp.float32),
            pltpu.SemaphoreType.DMA,
            pltpu.SemaphoreType.DMA,
            pltpu.SemaphoreType.DMA,
            pltpu.SemaphoreType.DMA,
        ],
    )


def _conv_body(x1_ref, x2_ref, f1s_ref, f1c_ref, g2f_ref, g2b_ref,
               twr_ref, twi_ref, y_ref, p_ref):
    f1s = f1s_ref[...]
    f1c = f1c_ref[...]
    g2f = g2f_ref[...]
    g2b = g2b_ref[...]
    twr = twr_ref[...][:, None, :]
    twi = twi_ref[...][:, None, :]

    def fwd(xt):
        y = jnp.dot(f1s, xt, preferred_element_type=jnp.float32)
        yr = y[:_N1].reshape(_N1, _R, _N2)
        yi = y[_N1:].reshape(_N1, _R, _N2)
        zr = (yr * twr - yi * twi).reshape(_N1 * _R, _N2)
        zi = (yr * twi + yi * twr).reshape(_N1 * _R, _N2)
        zc = jnp.concatenate([zr, zi], axis=1)
        return jnp.dot(zc, g2f, preferred_element_type=jnp.float32)

    u1 = fwd(x1_ref[...])
    u2 = fwd(x2_ref[...])
    u1r, u1i = u1[:, :_N2], u1[:, _N2:]
    u2r, u2i = u2[:, :_N2], u2[:, _N2:]
    pc = jnp.concatenate([u1r * u2r - u1i * u2i,
                          u1r * u2i + u1i * u2r], axis=1)
    s = jnp.dot(pc, g2b, preferred_element_type=jnp.float32)
    sr = s[:, :_N2].reshape(_N1, _R, _N2)
    si = s[:, _N2:].reshape(_N1, _R, _N2)
    tr = (sr * twr + si * twi).reshape(_N1, _R * _N2)
    ti = (si * twr - sr * twi).reshape(_N1, _R * _N2)
    tc = jnp.concatenate([tr, ti], axis=0)
    xo = jnp.dot(f1c, tc, preferred_element_type=jnp.float32) * (1.0 / _D)
    ax = jnp.abs(xo)
    y_ref[...] = jnp.sign(xo) * jnp.sqrt(ax)
    p_ref[...] = jnp.full((1, 1, 128), jnp.sum(ax) * (1.0 / 128.0), jnp.float32)


_conv = pl.pallas_call(
    _conv_body,
    grid=(_G,),
    in_specs=[
        pl.BlockSpec((_N1, _R * _N2), lambda g: (0, g)),
        pl.BlockSpec((_N1, _R * _N2), lambda g: (0, g)),
        pl.BlockSpec((2 * _N1, _N1), lambda g: (0, 0)),
        pl.BlockSpec((_N1, 2 * _N1), lambda g: (0, 0)),
        pl.BlockSpec((2 * _N2, 2 * _N2), lambda g: (0, 0)),
        pl.BlockSpec((2 * _N2, 2 * _N2), lambda g: (0, 0)),
        pl.BlockSpec((_N1, _N2), lambda g: (0, 0)),
        pl.BlockSpec((_N1, _N2), lambda g: (0, 0)),
    ],
    out_specs=[
        pl.BlockSpec((_N1, _R * _N2), lambda g: (0, g)),
        pl.BlockSpec((1, 1, 128), lambda g: (g, 0, 0)),
    ],
    out_shape=[jax.ShapeDtypeStruct((_N1, _B * _N2), jnp.float32),
               jax.ShapeDtypeStruct((_G, 1, 128), jnp.float32)],
)


def _scale_body(y_ref, p_ref, o_ref):
    total = jnp.sum(p_ref[...])
    norm = jnp.sqrt(jnp.maximum(total, 1e-12))
    y = y_ref[...] * (1.0 / norm)
    y = y.reshape(_N1, _R, _N2).transpose(1, 0, 2)
    o_ref[...] = y.reshape(_R, _D)


_scale = pl.pallas_call(
    _scale_body,
    grid=(_G,),
    in_specs=[
        pl.BlockSpec((_N1, _R * _N2), lambda g: (0, g)),
        pl.BlockSpec((_G, 1, 128), lambda g: (0, 0, 0)),
    ],
    out_specs=pl.BlockSpec((_R, _D), lambda g: (g, 0)),
    out_shape=jax.ShapeDtypeStruct((_B, _D), jnp.float32),
)


def _prep(h, s):
    order = jnp.argsort(h, stable=True)
    hs = h[order]
    first = jnp.concatenate([jnp.ones((1,), bool), hs[1:] != hs[:-1]])
    nf = jnp.sum(first.astype(jnp.int32))
    perm = order[jnp.argsort((~first).astype(jnp.int32), stable=True)]
    pad = (-nf) % _L
    i = jnp.arange(_NP)
    in_pad = (i >= nf) & (i < nf + pad)
    in_tail = i >= nf + pad + (_N - nf)
    srci = jnp.where(i < nf, i, jnp.where(in_pad, nf - 1, i - pad))
    srci = jnp.clip(srci, 0, _N - 1)
    hp = h[perm][srci]
    vp = perm[srci].astype(jnp.int32)
    sp = jnp.where(in_tail, 0.0, s[perm][srci])
    nf16 = (nf + pad) // _L
    return hp, vp, sp, nf16


def kernel(v1, v2, s1, s2, h1, h2):
    hp1, vp1, sp1, nf1 = _prep(h1, s1)
    hp2, vp2, sp2, nf2 = _prep(h2, s2)
    nfs = jnp.zeros((_L,), jnp.int32).at[0].set(nf1).at[1].set(nf2)
    skt1, skt2 = _sc_sketch()(v1, v2, hp1, vp1, sp1, hp2, vp2, sp2, nfs)
    y, parts = _conv(
        skt1, skt2,
        jnp.asarray(_F1S), jnp.asarray(_F1C),
        jnp.asarray(_G2F), jnp.asarray(_G2B),
        jnp.asarray(_TWR), jnp.asarray(_TWI),
    )
    return _scale(y, parts)


# speedup vs baseline: 1.0795x; 1.0795x over previous
import functools

import numpy as np
import jax
import jax.numpy as jnp
from jax import lax
from jax.experimental import pallas as pl
from jax.experimental.pallas import tpu as pltpu
from jax.experimental.pallas import tpu_sc as plsc

_B, _N, _D = 4096, 2048, 8192
_N1, _N2 = 64, 128
_NC, _NS = 2, 16
_NW = _NC * _NS
_RPW = _B // _NW
_L = 16
_NP = _N + _L

_R = 128
_G = _B // _R


def _dft(n):
    k = np.arange(n)
    ang = -2.0 * np.pi * np.outer(k, k) / n
    return np.cos(ang).astype(np.float32), np.sin(ang).astype(np.float32)


_F1R, _F1I = _dft(_N1)
_F2R, _F2I = _dft(_N2)
_F1S = np.vstack([_F1R, _F1I])
_F1C = np.hstack([_F1R, _F1I]) / np.float32(_D)
_G2F = np.block([[_F2R, _F2I], [-_F2I, _F2R]])
_G2B = np.block([[_F2R, -_F2I], [_F2I, _F2R]])
_ang = -2.0 * np.pi * np.outer(np.arange(_N1), np.arange(_N2)) / _D
_TWR = np.cos(_ang).astype(np.float32)
_TWI = np.sin(_ang).astype(np.float32)


def _sc_body(v1h, v2h, s1h, s2h, h1h, h2h, sk1h, sk2h,
             hv1, hv2, sv1, sv2,
             vb0, vb1, acc0, acc1,
             semv0, semv1, sema0, sema1):
    wid = lax.axis_index("s") * _NC + lax.axis_index("c")
    base = wid * _RPW
    pltpu.sync_copy(h1h, hv1)
    pltpu.sync_copy(h2h, hv2)
    pltpu.sync_copy(s1h, sv1)
    pltpu.sync_copy(s2h, sv2)
    vb = (vb0, vb1)
    acc = (acc0, acc1)
    semv = (semv0, semv1)
    sema = (sema0, sema1)
    zv = jnp.zeros((_L,), jnp.int32)

    for k in (0, 1):
        @pl.loop(0, _N1, unroll=4)
        def _z0(i, _k=k):
            for j in range(_N2 // _L):
                acc[_k][i, pl.ds(j * _L, _L)] = jnp.zeros((_L,), jnp.float32)

    def zero_touched(accr, hv):
        @pl.loop(0, _N // _L, unroll=8)
        def _z(j):
            idx = hv[pl.ds(j * _L, _L)]
            plsc.store_scatter(accr, [zv, idx], jnp.zeros((_L,), jnp.float32))

    def phase(vh, skh, hv, sv):
        for k in (0, 1):
            pltpu.make_async_copy(vh.at[base + k], vb[k], semv[k]).start()

        @pl.loop(0, _RPW // 2)
        def _pair(p):
            r0 = p * 2
            for k in (0, 1):
                r = r0 + k
                row = base + r

                @pl.when(r >= 2)
                def _reclaim(_k=k, _row=row):
                    pltpu.make_async_copy(
                        acc[_k],
                        skh.at[:, pl.ds((_row - 2) * _N2, _N2)],
                        sema[_k]).wait()
                    zero_touched(acc[_k], hv)

                pltpu.make_async_copy(vh.at[row], vb[k], semv[k]).wait()

                @pl.loop(0, _N // _L, unroll=8)
                def _scat(j, _k=k):
                    idx = hv[pl.ds(j * _L, _L)]
                    val = vb[_k][pl.ds(j * _L, _L)] * sv[pl.ds(j * _L, _L)]
                    plsc.addupdate_scatter(acc[_k], [zv, idx], val)

                pltpu.make_async_copy(acc[k],
                                      skh.at[:, pl.ds(row * _N2, _N2)],
                                      sema[k]).start()

                @pl.when(r + 2 < _RPW)
                def _prefetch(_k=k, _row=row):
                    pltpu.make_async_copy(vh.at[_row + 2], vb[_k],
                                          semv[_k]).start()

        for k in (0, 1):
            row = base + _RPW - 2 + k
            pltpu.make_async_copy(acc[k],
                                  skh.at[:, pl.ds(row * _N2, _N2)],
                                  sema[k]).wait()
            zero_touched(acc[k], hv)

    phase(v1h, sk1h, hv1, sv1)
    phase(v2h, sk2h, hv2, sv2)


@functools.cache
def _sc_sketch():
    return pl.kernel(
        _sc_body,
        out_type=[jax.ShapeDtypeStruct((_N1, _B * _N2), jnp.float32),
                  jax.ShapeDtypeStruct((_N1, _B * _N2), jnp.float32)],
        mesh=plsc.VectorSubcoreMesh(core_axis_name="c", subcore_axis_name="s",
                                    num_cores=_NC, num_subcores=_NS),
        compiler_params=pltpu.CompilerParams(needs_layout_passes=False),
        scratch_types=[
            pltpu.VMEM((_N,), jnp.int32),
            pltpu.VMEM((_N,), jnp.int32),
            pltpu.VMEM((_N,), jnp.float32),
            pltpu.VMEM((_N,), jnp.float32),
            pltpu.VMEM((_N,), jnp.float32),
            pltpu.VMEM((_N,), jnp.float32),
            pltpu.VMEM((_N1, _N2), jnp.float32),
            pltpu.VMEM((_N1, _N2), jnp.float32),
            pltpu.SemaphoreType.DMA,
            pltpu.SemaphoreType.DMA,
            pltpu.SemaphoreType.DMA,
            pltpu.SemaphoreType.DMA,
        ],
    )


def _conv_body(x1_ref, x2_ref, f1s_ref, f1c_ref, g2f_ref, g2b_ref,
               twr_ref, twi_ref, y_ref, p_ref):
    f1s = f1s_ref[...]
    f1c = f1c_ref[...]
    g2f = g2f_ref[...]
    g2b = g2b_ref[...]
    twr = twr_ref[...][:, None, :]
    twi = twi_ref[...][:, None, :]

    def fwd(xt):
        y = jnp.dot(f1s, xt, preferred_element_type=jnp.float32)
        yr = y[:_N1].reshape(_N1, _R, _N2)
        yi = y[_N1:].reshape(_N1, _R, _N2)
        zr = (yr * twr - yi * twi).reshape(_N1 * _R, _N2)
        zi = (yr * twi + yi * twr).reshape(_N1 * _R, _N2)
        zc = jnp.concatenate([zr, zi], axis=1)
        return jnp.dot(zc, g2f, preferred_element_type=jnp.float32)

    u1 = fwd(x1_ref[...])
    u2 = fwd(x2_ref[...])
    u1r, u1i = u1[:, :_N2], u1[:, _N2:]
    u2r, u2i = u2[:, :_N2], u2[:, _N2:]
    pc = jnp.concatenate([u1r * u2r - u1i * u2i,
                          u1r * u2i + u1i * u2r], axis=1)
    s = jnp.dot(pc, g2b, preferred_element_type=jnp.float32)
    sr = s[:, :_N2].reshape(_N1, _R, _N2)
    si = s[:, _N2:].reshape(_N1, _R, _N2)
    tr = (sr * twr + si * twi).reshape(_N1, _R * _N2)
    ti = (si * twr - sr * twi).reshape(_N1, _R * _N2)
    tc = jnp.concatenate([tr, ti], axis=0)
    xo = jnp.dot(f1c, tc, preferred_element_type=jnp.float32)
    ax = jnp.abs(xo)
    y_ref[...] = jnp.sign(xo) * jnp.sqrt(ax)
    p_ref[...] = jnp.full((1, 1, 128), jnp.sum(ax) * (1.0 / 128.0), jnp.float32)


_conv = pl.pallas_call(
    _conv_body,
    grid=(_G,),
    in_specs=[
        pl.BlockSpec((_N1, _R * _N2), lambda g: (0, g)),
        pl.BlockSpec((_N1, _R * _N2), lambda g: (0, g)),
        pl.BlockSpec((2 * _N1, _N1), lambda g: (0, 0)),
        pl.BlockSpec((_N1, 2 * _N1), lambda g: (0, 0)),
        pl.BlockSpec((2 * _N2, 2 * _N2), lambda g: (0, 0)),
        pl.BlockSpec((2 * _N2, 2 * _N2), lambda g: (0, 0)),
        pl.BlockSpec((_N1, _N2), lambda g: (0, 0)),
        pl.BlockSpec((_N1, _N2), lambda g: (0, 0)),
    ],
    out_specs=[
        pl.BlockSpec((_N1, _R * _N2), lambda g: (0, g)),
        pl.BlockSpec((1, 1, 128), lambda g: (g, 0, 0)),
    ],
    out_shape=[jax.ShapeDtypeStruct((_N1, _B * _N2), jnp.float32),
               jax.ShapeDtypeStruct((_G, 1, 128), jnp.float32)],
)


def _scale_body(y_ref, p_ref, o_ref):
    total = jnp.sum(p_ref[...])
    norm = jnp.sqrt(jnp.maximum(total, 1e-12))
    y = y_ref[...] * (1.0 / norm)
    y = y.reshape(_N1, _R, _N2).transpose(1, 0, 2)
    o_ref[...] = y.reshape(_R, _D)


_scale = pl.pallas_call(
    _scale_body,
    grid=(_G,),
    in_specs=[
        pl.BlockSpec((_N1, _R * _N2), lambda g: (0, g)),
        pl.BlockSpec((_G, 1, 128), lambda g: (0, 0, 0)),
    ],
    out_specs=pl.BlockSpec((_R, _D), lambda g: (g, 0)),
    out_shape=jax.ShapeDtypeStruct((_B, _D), jnp.float32),
)


def kernel(v1, v2, s1, s2, h1, h2):
    skt1, skt2 = _sc_sketch()(v1, v2, s1, s2, h1, h2)
    y, parts = _conv(
        skt1, skt2,
        jnp.asarray(_F1S), jnp.asarray(_F1C),
        jnp.asarray(_G2F), jnp.asarray(_G2B),
        jnp.asarray(_TWR), jnp.asarray(_TWI),
    )
    return _scale(y, parts)


# speedup vs baseline: 1.0813x; 1.0016x over previous
import functools

import numpy as np
import jax
import jax.numpy as jnp
from jax import lax
from jax.experimental import pallas as pl
from jax.experimental.pallas import tpu as pltpu
from jax.experimental.pallas import tpu_sc as plsc

_B, _N, _D = 4096, 2048, 8192
_N1, _N2 = 64, 128
_NC, _NS = 2, 16
_NW = _NC * _NS
_RPW = _B // _NW
_L = 16
_NP = _N + _L

_R = 128
_G = _B // _R


def _dft(n):
    k = np.arange(n)
    ang = -2.0 * np.pi * np.outer(k, k) / n
    return np.cos(ang).astype(np.float32), np.sin(ang).astype(np.float32)


_F1R, _F1I = _dft(_N1)
_F2R, _F2I = _dft(_N2)
_F1S = np.vstack([_F1R, _F1I])
_F1C = np.hstack([_F1R, _F1I]) / np.float32(_D)
_G2F = np.block([[_F2R, _F2I], [-_F2I, _F2R]])
_G2B = np.block([[_F2R, -_F2I], [_F2I, _F2R]])
_ang = -2.0 * np.pi * np.outer(np.arange(_N1), np.arange(_N2)) / _D
_TWR = np.cos(_ang).astype(np.float32)
_TWI = np.sin(_ang).astype(np.float32)


def _sc_body(v1h, v2h, s1h, s2h, h1h, h2h, sk1h, sk2h,
             hv1, hv2, sv1, sv2,
             vb0, vb1, acc0, acc1,
             semv0, semv1, sema0, sema1):
    wid = lax.axis_index("s") * _NC + lax.axis_index("c")
    base = wid * _RPW
    pltpu.sync_copy(h1h, hv1)
    pltpu.sync_copy(h2h, hv2)
    pltpu.sync_copy(s1h, sv1)
    pltpu.sync_copy(s2h, sv2)
    vb = (vb0, vb1)
    acc = (acc0, acc1)
    semv = (semv0, semv1)
    sema = (sema0, sema1)
    zv = jnp.zeros((_L,), jnp.int32)

    for k in (0, 1):
        @pl.loop(0, _N1, unroll=4)
        def _z0(i, _k=k):
            for j in range(_N2 // _L):
                acc[_k][i, pl.ds(j * _L, _L)] = jnp.zeros((_L,), jnp.float32)

    def zero_touched(accr, hv):
        @pl.loop(0, _N // _L, unroll=16)
        def _z(j):
            idx = hv[pl.ds(j * _L, _L)]
            plsc.store_scatter(accr, [zv, idx], jnp.zeros((_L,), jnp.float32))

    def phase(vh, skh, hv, sv):
        for k in (0, 1):
            pltpu.make_async_copy(vh.at[base + k], vb[k], semv[k]).start()

        @pl.loop(0, _RPW // 2)
        def _pair(p):
            r0 = p * 2
            for k in (0, 1):
                r = r0 + k
                row = base + r

                @pl.when(r >= 2)
                def _reclaim(_k=k, _row=row):
                    pltpu.make_async_copy(
                        acc[_k],
                        skh.at[:, pl.ds((_row - 2) * _N2, _N2)],
                        sema[_k]).wait()
                    zero_touched(acc[_k], hv)

                pltpu.make_async_copy(vh.at[row], vb[k], semv[k]).wait()

                @pl.loop(0, _N // _L, unroll=16)
                def _scat(j, _k=k):
                    idx = hv[pl.ds(j * _L, _L)]
                    val = vb[_k][pl.ds(j * _L, _L)] * sv[pl.ds(j * _L, _L)]
                    plsc.addupdate_scatter(acc[_k], [zv, idx], val)

                pltpu.make_async_copy(acc[k],
                                      skh.at[:, pl.ds(row * _N2, _N2)],
                                      sema[k]).start()

                @pl.when(r + 2 < _RPW)
                def _prefetch(_k=k, _row=row):
                    pltpu.make_async_copy(vh.at[_row + 2], vb[_k],
                                          semv[_k]).start()

        for k in (0, 1):
            row = base + _RPW - 2 + k
            pltpu.make_async_copy(acc[k],
                                  skh.at[:, pl.ds(row * _N2, _N2)],
                                  sema[k]).wait()
            zero_touched(acc[k], hv)

    phase(v1h, sk1h, hv1, sv1)
    phase(v2h, sk2h, hv2, sv2)


@functools.cache
def _sc_sketch():
    return pl.kernel(
        _sc_body,
        out_type=[jax.ShapeDtypeStruct((_N1, _B * _N2), jnp.float32),
                  jax.ShapeDtypeStruct((_N1, _B * _N2), jnp.float32)],
        mesh=plsc.VectorSubcoreMesh(core_axis_name="c", subcore_axis_name="s",
                                    num_cores=_NC, num_subcores=_NS),
        compiler_params=pltpu.CompilerParams(needs_layout_passes=False),
        scratch_types=[
            pltpu.VMEM((_N,), jnp.int32),
            pltpu.VMEM((_N,), jnp.int32),
            pltpu.VMEM((_N,), jnp.float32),
            pltpu.VMEM((_N,), jnp.float32),
            pltpu.VMEM((_N,), jnp.float32),
            pltpu.VMEM((_N,), jnp.float32),
            pltpu.VMEM((_N1, _N2), jnp.float32),
            pltpu.VMEM((_N1, _N2), jnp.float32),
            pltpu.SemaphoreType.DMA,
            pltpu.SemaphoreType.DMA,
            pltpu.SemaphoreType.DMA,
            pltpu.SemaphoreType.DMA,
        ],
    )


def _conv_body(x1_ref, x2_ref, f1s_ref, f1c_ref, g2f_ref, g2b_ref,
               twr_ref, twi_ref, y_ref, p_ref):
    f1s = f1s_ref[...]
    f1c = f1c_ref[...]
    g2f = g2f_ref[...]
    g2b = g2b_ref[...]
    twr = twr_ref[...][:, None, :]
    twi = twi_ref[...][:, None, :]

    def fwd(xt):
        y = jnp.dot(f1s, xt, preferred_element_type=jnp.float32)
        yr = y[:_N1].reshape(_N1, _R, _N2)
        yi = y[_N1:].reshape(_N1, _R, _N2)
        zr = (yr * twr - yi * twi).reshape(_N1 * _R, _N2)
        zi = (yr * twi + yi * twr).reshape(_N1 * _R, _N2)
        zc = jnp.concatenate([zr, zi], axis=1)
        return jnp.dot(zc, g2f, preferred_element_type=jnp.float32)

    u1 = fwd(x1_ref[...])
    u2 = fwd(x2_ref[...])
    u1r, u1i = u1[:, :_N2], u1[:, _N2:]
    u2r, u2i = u2[:, :_N2], u2[:, _N2:]
    pc = jnp.concatenate([u1r * u2r - u1i * u2i,
                          u1r * u2i + u1i * u2r], axis=1)
    s = jnp.dot(pc, g2b, preferred_element_type=jnp.float32)
    sr = s[:, :_N2].reshape(_N1, _R, _N2)
    si = s[:, _N2:].reshape(_N1, _R, _N2)
    tr = (sr * twr + si * twi).reshape(_N1, _R * _N2)
    ti = (si * twr - sr * twi).reshape(_N1, _R * _N2)
    tc = jnp.concatenate([tr, ti], axis=0)
    xo = jnp.dot(f1c, tc, preferred_element_type=jnp.float32)
    ax = jnp.abs(xo)
    y_ref[...] = jnp.sign(xo) * jnp.sqrt(ax)
    p_ref[...] = jnp.full((1, 1, 128), jnp.sum(ax) * (1.0 / 128.0), jnp.float32)


_conv = pl.pallas_call(
    _conv_body,
    grid=(_G,),
    in_specs=[
        pl.BlockSpec((_N1, _R * _N2), lambda g: (0, g)),
        pl.BlockSpec((_N1, _R * _N2), lambda g: (0, g)),
        pl.BlockSpec((2 * _N1, _N1), lambda g: (0, 0)),
        pl.BlockSpec((_N1, 2 * _N1), lambda g: (0, 0)),
        pl.BlockSpec((2 * _N2, 2 * _N2), lambda g: (0, 0)),
        pl.BlockSpec((2 * _N2, 2 * _N2), lambda g: (0, 0)),
        pl.BlockSpec((_N1, _N2), lambda g: (0, 0)),
        pl.BlockSpec((_N1, _N2), lambda g: (0, 0)),
    ],
    out_specs=[
        pl.BlockSpec((_N1, _R * _N2), lambda g: (0, g)),
        pl.BlockSpec((1, 1, 128), lambda g: (g, 0, 0)),
    ],
    out_shape=[jax.ShapeDtypeStruct((_N1, _B * _N2), jnp.float32),
               jax.ShapeDtypeStruct((_G, 1, 128), jnp.float32)],
)


def _scale_body(y_ref, p_ref, o_ref):
    total = jnp.sum(p_ref[...])
    norm = jnp.sqrt(jnp.maximum(total, 1e-12))
    y = y_ref[...] * (1.0 / norm)
    y = y.reshape(_N1, _R, _N2).transpose(1, 0, 2)
    o_ref[...] = y.reshape(_R, _D)


_scale = pl.pallas_call(
    _scale_body,
    grid=(_G,),
    in_specs=[
        pl.BlockSpec((_N1, _R * _N2), lambda g: (0, g)),
        pl.BlockSpec((_G, 1, 128), lambda g: (0, 0, 0)),
    ],
    out_specs=pl.BlockSpec((_R, _D), lambda g: (g, 0)),
    out_shape=jax.ShapeDtypeStruct((_B, _D), jnp.float32),
)


def kernel(v1, v2, s1, s2, h1, h2):
    skt1, skt2 = _sc_sketch()(v1, v2, s1, s2, h1, h2)
    y, parts = _conv(
        skt1, skt2,
        jnp.asarray(_F1S), jnp.asarray(_F1C),
        jnp.asarray(_G2F), jnp.asarray(_G2B),
        jnp.asarray(_TWR), jnp.asarray(_TWI),
    )
    return _scale(y, parts)
